# baseline (speedup 1.0000x reference)
import dataclasses

import numpy as np

import jax
import jax.numpy as jnp
from jax import lax
from jax.experimental import pallas as pl
from jax.experimental.pallas import tpu as pltpu
from jax.experimental.pallas import tpu_sc as plsc

BATCH = 4096
SEQ = 200
VOCAB_ROWS = 1000000
EMBED = 32
NUM_IDS = BATCH * SEQ

PACK = 128 // EMBED
LANES = 16

NUM_CORES = 2
NUM_SUBCORES = 16
NUM_WORKERS = NUM_CORES * NUM_SUBCORES
IDS_PER_WORKER = NUM_IDS // NUM_WORKERS

CHUNK = 128
STEPS = IDS_PER_WORKER // CHUNK


def kernel(input_ids, table):
    idx = input_ids.reshape(NUM_IDS)
    qidx = idx >> 2
    packed = table.reshape(VOCAB_ROWS // PACK, EMBED * PACK)
    mesh = plsc.VectorSubcoreMesh(core_axis_name="c", subcore_axis_name="s")

    cp = pltpu.CompilerParams()
    if "needs_layout_passes" in pltpu.CompilerParams.__dataclass_fields__:
        cp = dataclasses.replace(cp, needs_layout_passes=False)

    @pl.kernel(
        out_type=jax.ShapeDtypeStruct((NUM_IDS, EMBED), table.dtype),
        mesh=mesh,
        compiler_params=cp,
        scratch_types=[
            pltpu.VMEM((CHUNK,), jnp.int32),
            pltpu.VMEM((CHUNK,), jnp.int32),
            pltpu.VMEM((CHUNK, EMBED * PACK), jnp.float32),
            pltpu.VMEM((CHUNK, EMBED), jnp.float32),
            pltpu.SemaphoreType.DMA,
        ],
    )
    def sc_gather(
        packed_hbm, qidx_hbm, idx_hbm, out_hbm, q_v, id_v, rows_v, out_v, sem
    ):
        wid = lax.axis_index("s") * NUM_CORES + lax.axis_index("c")
        base = wid * IDS_PER_WORKER
        iota16 = lax.iota(jnp.int32, LANES)

        @pl.loop(0, STEPS)
        def _(j):
            off = base + j * CHUNK
            pltpu.sync_copy(qidx_hbm.at[pl.ds(off, CHUNK)], q_v)
            pltpu.sync_copy(idx_hbm.at[pl.ds(off, CHUNK)], id_v)
            pltpu.async_copy(packed_hbm.at[q_v], rows_v, sem).wait()
            for g in range(CHUNK // LANES):
                row16 = iota16 + (g * LANES)
                ids16 = id_v[pl.ds(g * LANES, LANES)]
                colb16 = (ids16 & 3) << 5
                for c in range(EMBED):
                    vals = plsc.load_gather(rows_v, [row16, colb16 + c])
                    plsc.store_scatter(
                        out_v,
                        [row16, jnp.full((LANES,), c, jnp.int32)],
                        vals,
                    )
            pltpu.sync_copy(out_v, out_hbm.at[pl.ds(off, CHUNK)])

    out = sc_gather(packed, qidx, idx)
    return out.reshape(BATCH, SEQ, EMBED)


# speedup vs baseline: 1.2702x; 1.2702x over previous
import dataclasses

import numpy as np

import jax
import jax.numpy as jnp
from jax import lax
from jax.experimental import pallas as pl
from jax.experimental.pallas import tpu as pltpu
from jax.experimental.pallas import tpu_sc as plsc

BATCH = 4096
SEQ = 200
VOCAB_ROWS = 1000000
EMBED = 32
NUM_IDS = BATCH * SEQ

PACK = 128 // EMBED
LANES = 16

NUM_CORES = 2
NUM_SUBCORES = 16
NUM_WORKERS = NUM_CORES * NUM_SUBCORES
IDS_PER_WORKER = NUM_IDS // NUM_WORKERS

CHUNK = 128
GATHER = 128
STEPS = IDS_PER_WORKER // CHUNK
GROUPS = CHUNK // LANES
PAD = 2 * CHUNK


def kernel(input_ids, table):
    idx = input_ids.reshape(NUM_IDS)
    pad = jnp.zeros((PAD,), jnp.int32)
    qidx = jnp.concatenate([idx >> 2, pad])
    colb = jnp.concatenate([(idx & 3) << 5, pad])
    packed = table.reshape(VOCAB_ROWS // PACK, EMBED * PACK)
    mesh = plsc.VectorSubcoreMesh(core_axis_name="c", subcore_axis_name="s")

    cp = pltpu.CompilerParams()
    if "needs_layout_passes" in pltpu.CompilerParams.__dataclass_fields__:
        cp = dataclasses.replace(cp, needs_layout_passes=False)

    @pl.kernel(
        out_type=jax.ShapeDtypeStruct((NUM_IDS, EMBED), table.dtype),
        mesh=mesh,
        compiler_params=cp,
        scratch_types=[
            pltpu.VMEM((CHUNK,), jnp.int32),
            pltpu.VMEM((CHUNK,), jnp.int32),
            pltpu.VMEM((CHUNK,), jnp.int32),
            pltpu.VMEM((CHUNK,), jnp.int32),
            pltpu.VMEM((CHUNK, EMBED * PACK), jnp.float32),
            pltpu.VMEM((CHUNK, EMBED * PACK), jnp.float32),
            pltpu.VMEM((CHUNK, EMBED), jnp.float32),
            pltpu.VMEM((CHUNK, EMBED), jnp.float32),
            pltpu.SemaphoreType.DMA,
            pltpu.SemaphoreType.DMA,
            pltpu.SemaphoreType.DMA,
            pltpu.SemaphoreType.DMA,
            pltpu.SemaphoreType.DMA,
            pltpu.SemaphoreType.DMA,
        ],
    )
    def sc_gather(
        packed_hbm,
        qidx_hbm,
        colb_hbm,
        out_hbm,
        qbuf_a,
        qbuf_b,
        cbuf_a,
        cbuf_b,
        rows_a,
        rows_b,
        outb_a,
        outb_b,
        semi_a,
        semi_b,
        semg_a,
        semg_b,
        semw_a,
        semw_b,
    ):
        qbuf = (qbuf_a, qbuf_b)
        cbuf = (cbuf_a, cbuf_b)
        rows = (rows_a, rows_b)
        outb = (outb_a, outb_b)
        semi = (semi_a, semi_b)
        semg = (semg_a, semg_b)
        semw = (semw_a, semw_b)
        wid = lax.axis_index("s") * NUM_CORES + lax.axis_index("c")
        base = wid * IDS_PER_WORKER
        iota16 = lax.iota(jnp.int32, LANES)

        def start_idx(c, u):
            off = base + c * CHUNK
            pltpu.async_copy(
                qidx_hbm.at[pl.ds(off, CHUNK)], qbuf[u], semi[u]
            )
            pltpu.async_copy(
                colb_hbm.at[pl.ds(off, CHUNK)], cbuf[u], semi[u]
            )

        def wait_idx(u):
            pltpu.make_async_copy(
                qidx_hbm.at[pl.ds(0, CHUNK)], qbuf[u], semi[u]
            ).wait()
            pltpu.make_async_copy(
                colb_hbm.at[pl.ds(0, CHUNK)], cbuf[u], semi[u]
            ).wait()

        def start_gather(u):
            for h in range(CHUNK // GATHER):
                pltpu.async_copy(
                    packed_hbm.at[qbuf[u].at[pl.ds(h * GATHER, GATHER)]],
                    rows[u].at[pl.ds(h * GATHER, GATHER)],
                    semg[u],
                )

        def wait_gather(u):
            for h in range(CHUNK // GATHER):
                pltpu.make_async_copy(
                    packed_hbm.at[pl.ds(0, GATHER)],
                    rows[u].at[pl.ds(h * GATHER, GATHER)],
                    semg[u],
                ).wait()

        def extract(u):
            rows_u = rows[u]
            outb_u = outb[u]
            cbuf_u = cbuf[u]

            for g in range(GROUPS):
                row16 = iota16 + g * LANES
                colb16 = cbuf_u[pl.ds(g * LANES, LANES)]
                for c in range(EMBED):
                    vals = plsc.load_gather(rows_u, [row16, colb16 + c])
                    plsc.store_scatter(
                        outb_u,
                        [row16, jnp.full((LANES,), c, jnp.int32)],
                        vals,
                    )

        def start_write(c, u):
            off = base + c * CHUNK
            pltpu.async_copy(
                outb[u], out_hbm.at[pl.ds(off, CHUNK)], semw[u]
            )

        def wait_write(u):
            pltpu.make_async_copy(
                outb[u], out_hbm.at[pl.ds(0, CHUNK)], semw[u]
            ).wait()

        start_idx(0, 0)
        wait_idx(0)
        start_gather(0)
        start_idx(1, 1)

        @pl.loop(0, STEPS // 2)
        def _(j):
            c0 = j * 2
            for c, u, v in ((c0, 0, 1), (c0 + 1, 1, 0)):
                wait_gather(u)
                wait_idx(v)
                start_gather(v)

                @pl.when(j > 0)
                def _():
                    wait_write(u)

                extract(u)
                start_write(c, u)
                start_idx(c + 2, u)

        wait_gather(0)
        wait_idx(1)
        wait_write(0)
        wait_write(1)

    out = sc_gather(packed, qidx, colb)
    return out.reshape(BATCH, SEQ, EMBED)


# speedup vs baseline: 2.0869x; 1.6431x over previous
import jax
import jax.numpy as jnp
from jax import lax
from jax.experimental import pallas as pl
from jax.experimental.pallas import tpu as pltpu
from jax.experimental.pallas import tpu_sc as plsc

BATCH = 4096
SEQ = 200
VOCAB_ROWS = 1000000
EMBED = 32
NUM_IDS = BATCH * SEQ

NUM_CORES = 2
NUM_SUBCORES = 16
NUM_WORKERS = NUM_CORES * NUM_SUBCORES
IDS_PER_WORKER = NUM_IDS // NUM_WORKERS
SAMPLES_PER_WORKER = BATCH // NUM_WORKERS

GATHER_SPLITS = ((0, 128), (128, 72))
STEPS = SAMPLES_PER_WORKER


def kernel(input_ids, table):
    idx = input_ids.reshape(NUM_IDS)
    packed = lax.optimization_barrier(table.reshape(VOCAB_ROWS // 4, 128))
    tlin = packed.reshape(VOCAB_ROWS, EMBED)
    mesh = plsc.VectorSubcoreMesh(core_axis_name="c", subcore_axis_name="s")

    cp = pltpu.CompilerParams(use_tc_tiling_on_sc=False)

    @pl.kernel(
        out_type=jax.ShapeDtypeStruct((BATCH, SEQ, EMBED), table.dtype),
        mesh=mesh,
        compiler_params=cp,
        scratch_types=[
            pltpu.VMEM((SEQ,), jnp.int32),
            pltpu.VMEM((SEQ,), jnp.int32),
            pltpu.VMEM((SEQ, EMBED), jnp.float32),
            pltpu.VMEM((SEQ, EMBED), jnp.float32),
            pltpu.SemaphoreType.DMA,
            pltpu.SemaphoreType.DMA,
            pltpu.SemaphoreType.DMA,
            pltpu.SemaphoreType.DMA,
            pltpu.SemaphoreType.DMA,
            pltpu.SemaphoreType.DMA,
        ],
    )
    def sc_gather(
        table_hbm,
        idx_hbm,
        out_hbm,
        ibuf_a,
        ibuf_b,
        rows_a,
        rows_b,
        semi_a,
        semi_b,
        semg_a,
        semg_b,
        semw_a,
        semw_b,
    ):
        ibuf = (ibuf_a, ibuf_b)
        rows = (rows_a, rows_b)
        semi = (semi_a, semi_b)
        semg = (semg_a, semg_b)
        semw = (semw_a, semw_b)
        wid = lax.axis_index("s") * NUM_CORES + lax.axis_index("c")
        base = wid * IDS_PER_WORKER
        sbase = wid * SAMPLES_PER_WORKER

        def start_idx(c, u):
            off = base + jnp.minimum(c, STEPS - 1) * SEQ
            pltpu.async_copy(idx_hbm.at[pl.ds(off, SEQ)], ibuf[u], semi[u])

        def wait_idx(u):
            pltpu.make_async_copy(
                idx_hbm.at[pl.ds(0, SEQ)], ibuf[u], semi[u]
            ).wait()

        def start_gather(u):
            for off, ln in GATHER_SPLITS:
                pltpu.async_copy(
                    table_hbm.at[ibuf[u].at[pl.ds(off, ln)]],
                    rows[u].at[pl.ds(off, ln)],
                    semg[u],
                )

        def wait_gather(u):
            for off, ln in GATHER_SPLITS:
                pltpu.make_async_copy(
                    table_hbm.at[pl.ds(0, ln)],
                    rows[u].at[pl.ds(off, ln)],
                    semg[u],
                ).wait()

        def start_write(c, u):
            pltpu.async_copy(rows[u], out_hbm.at[sbase + c], semw[u])

        def wait_write(u):
            pltpu.make_async_copy(
                rows[u], out_hbm.at[sbase], semw[u]
            ).wait()

        start_idx(0, 0)
        wait_idx(0)
        start_gather(0)
        start_idx(1, 1)

        @pl.loop(0, STEPS // 2)
        def _(j):
            c0 = j * 2
            wait_gather(0)
            wait_idx(1)

            @pl.when(j > 0)
            def _():
                wait_write(1)

            start_gather(1)
            start_write(c0, 0)
            start_idx(c0 + 2, 0)

            wait_gather(1)
            wait_idx(0)
            wait_write(0)
            start_gather(0)
            start_write(c0 + 1, 1)
            start_idx(c0 + 3, 1)

        wait_gather(0)
        wait_idx(1)
        wait_write(1)

    return sc_gather(tlin, idx)


# speedup vs baseline: 2.2133x; 1.0606x over previous
import dataclasses

import jax
import jax.numpy as jnp
from jax import lax
from jax.experimental import pallas as pl
from jax.experimental.pallas import tpu as pltpu
from jax.experimental.pallas import tpu_sc as plsc

BATCH = 4096
SEQ = 200
VOCAB_ROWS = 1000000
EMBED = 32
NUM_IDS = BATCH * SEQ
ROW_BYTES = EMBED * 4

NUM_CORES = 2
NUM_SUBCORES = 16
NUM_WORKERS = NUM_CORES * NUM_SUBCORES
IDS_PER_WORKER = NUM_IDS // NUM_WORKERS

CHUNK = 1280
GATHER = 128
STEPS = IDS_PER_WORKER // CHUNK


def kernel(input_ids, table):
    idx = input_ids.reshape(NUM_IDS)
    mesh = plsc.VectorSubcoreMesh(core_axis_name="c", subcore_axis_name="s")

    cp = pltpu.CompilerParams(use_tc_tiling_on_sc=False)

    @pl.kernel(
        out_type=jax.ShapeDtypeStruct((NUM_IDS, EMBED), table.dtype),
        mesh=mesh,
        compiler_params=cp,
        scratch_types=[
            pltpu.VMEM((CHUNK,), jnp.int32),
            pltpu.VMEM((CHUNK,), jnp.int32),
            pltpu.VMEM((CHUNK, EMBED), jnp.float32),
            pltpu.VMEM((CHUNK, EMBED), jnp.float32),
            pltpu.SemaphoreType.DMA,
            pltpu.SemaphoreType.DMA,
            pltpu.SemaphoreType.DMA,
            pltpu.SemaphoreType.DMA,
            pltpu.SemaphoreType.DMA,
            pltpu.SemaphoreType.DMA,
        ],
    )
    def sc_gather(
        table_hbm,
        idx_hbm,
        out_hbm,
        ibuf_a,
        ibuf_b,
        rows_a,
        rows_b,
        semi_a,
        semi_b,
        semg_a,
        semg_b,
        semw_a,
        semw_b,
    ):
        ibuf = (ibuf_a, ibuf_b)
        rows = (rows_a, rows_b)
        semi = (semi_a, semi_b)
        semg = (semg_a, semg_b)
        semw = (semw_a, semw_b)
        wid = lax.axis_index("s") * NUM_CORES + lax.axis_index("c")
        base = wid * IDS_PER_WORKER

        def start_idx(c, u):
            off = base + jnp.minimum(c, STEPS - 1) * CHUNK
            pltpu.async_copy(idx_hbm.at[pl.ds(off, CHUNK)], ibuf[u], semi[u])

        def wait_idx(u):
            pltpu.make_async_copy(
                idx_hbm.at[pl.ds(0, CHUNK)], ibuf[u], semi[u]
            ).wait()

        def start_gather(u):
            for h in range(CHUNK // GATHER):
                pltpu.async_copy(
                    table_hbm.at[ibuf[u].at[pl.ds(h * GATHER, GATHER)]],
                    rows[u].at[pl.ds(h * GATHER, GATHER)],
                    semg[u],
                )

        def wait_gather(u):
            for h in range(CHUNK // GATHER):
                pltpu.make_async_copy(
                    table_hbm.at[pl.ds(0, GATHER)],
                    rows[u].at[pl.ds(h * GATHER, GATHER)],
                    semg[u],
                ).wait()

        def start_write(c, u):
            off = base + c * CHUNK
            pltpu.async_copy(rows[u], out_hbm.at[pl.ds(off, CHUNK)], semw[u])

        def wait_write(u):
            pltpu.make_async_copy(
                rows[u], out_hbm.at[pl.ds(0, CHUNK)], semw[u]
            ).wait()

        start_idx(0, 0)
        wait_idx(0)
        start_gather(0)
        start_idx(1, 1)

        @pl.loop(0, STEPS // 2)
        def _(j):
            c0 = j * 2
            wait_gather(0)
            wait_idx(1)

            @pl.when(j > 0)
            def _():
                wait_write(1)

            start_gather(1)
            start_write(c0, 0)
            start_idx(c0 + 2, 0)

            wait_gather(1)
            wait_idx(0)
            wait_write(0)
            start_gather(0)
            start_write(c0 + 1, 1)
            start_idx(c0 + 3, 1)

        wait_gather(0)
        wait_idx(1)
        wait_write(1)

    out = sc_gather(table, idx)
    return out.reshape(BATCH, SEQ, EMBED)
